# baseline (speedup 1.0000x reference)
import jax
import jax.numpy as jnp
from jax import lax
from jax.experimental import pallas as pl
from jax.experimental.pallas import tpu as pltpu
from jax.experimental.pallas import tpu_sc as plsc

_S = 64.0
_M = 0.4

_NC = 2
_NS = 16
_NW = _NC * _NS
_CH = 3072
_C_SC = 98304
_TAIL_BLOCK = 2048
_UNROLL = 16


def _sc_scale(logits, labx):
    b, c = logits.shape
    rows_per_w = b // _NW
    nch = _C_SC // _CH
    ngrp = rows_per_w // 8
    t_total = ngrp * nch

    def body(logits_ref, labx_ref, out_ref,
             labx_v, in0, in1, out0, out1, ls0, ls1, ss0, ss1):
        cid = lax.axis_index("c")
        sid = lax.axis_index("s")
        wid = sid * _NC + cid
        r0 = wid * rows_per_w
        pltpu.sync_copy(
            labx_ref.at[pl.ds(pl.multiple_of(r0, 8), rows_per_w)], labx_v)

        ins = (in0, in1)
        outs = (out0, out1)
        lsems = (ls0, ls1)
        ssems = (ss0, ss1)

        def coords(t):
            rg = t // nch
            row8 = pl.multiple_of(r0 + rg * 8, 8)
            c0 = pl.multiple_of((t % nch) * _CH, 128)
            return rg, row8, c0

        def src_slice(t):
            _, row8, c0 = coords(t)
            return logits_ref.at[pl.ds(row8, 8), pl.ds(c0, _CH)]

        def dst_slice(t):
            _, row8, c0 = coords(t)
            return out_ref.at[pl.ds(row8, 8), pl.ds(c0, _CH)]

        pltpu.async_copy(src_slice(0), in0, ls0)
        pltpu.async_copy(src_slice(1), in1, ls1)

        lanes = lax.broadcasted_iota(jnp.int32, (16,), 0)

        def group(g, carry):
            for bb in range(2):
                t = g * 2 + bb
                ib, ob, ls, ss = ins[bb], outs[bb], lsems[bb], ssems[bb]
                pltpu.make_async_copy(src_slice(t), ib, ls).wait()

                rg, _, c0 = coords(t)
                for r in range(8):
                    rowloc = rg * 8 + r
                    labsplat = labx_v[rowloc, pl.ds(0, 16)]
                    lrel = labsplat - c0

                    def scale(i, carry2, r=r, lrel=lrel):
                        base = i * (16 * _UNROLL)
                        for u in range(_UNROLL):
                            sl = pl.ds(base + u * 16, 16)
                            m = (lrel - (base + u * 16)) == lanes
                            ob[r, sl] = (
                                ib[r, sl] - jnp.where(m, _M, 0.0)) * _S
                        return carry2

                    lax.fori_loop(0, _CH // (16 * _UNROLL), scale, 0)

                @pl.when(t >= 2)
                def _(ob=ob, ss=ss, t=t):
                    pltpu.make_async_copy(ob, dst_slice(t - 2), ss).wait()

                pltpu.async_copy(ob, dst_slice(t), ss)

                @pl.when(t + 2 < t_total)
                def _(ib=ib, ls=ls, t=t):
                    pltpu.async_copy(src_slice(t + 2), ib, ls)

            return carry

        lax.fori_loop(0, t_total // 2, group, 0)
        pltpu.make_async_copy(out0, dst_slice(t_total - 2), ss0).wait()
        pltpu.make_async_copy(out1, dst_slice(t_total - 1), ss1).wait()

    mesh = plsc.VectorSubcoreMesh(
        core_axis_name="c", subcore_axis_name="s",
        num_cores=_NC, num_subcores=_NS,
    )
    fn = pl.kernel(
        body,
        out_type=jax.ShapeDtypeStruct((b, c), jnp.float32),
        mesh=mesh,
        scratch_types=[
            pltpu.VMEM((rows_per_w, 128), jnp.int32),
            pltpu.VMEM((8, _CH), jnp.float32),
            pltpu.VMEM((8, _CH), jnp.float32),
            pltpu.VMEM((8, _CH), jnp.float32),
            pltpu.VMEM((8, _CH), jnp.float32),
            pltpu.SemaphoreType.DMA,
            pltpu.SemaphoreType.DMA,
            pltpu.SemaphoreType.DMA,
            pltpu.SemaphoreType.DMA,
        ],
    )
    return fn(logits, labx)


def _tail_body(alias_ref, labels_ref, x_ref, o_ref):
    del alias_ref
    bb, bc = x_ref.shape
    cols = _C_SC + jax.lax.broadcasted_iota(jnp.int32, (bb, bc), 1)
    lab = labels_ref[...]
    x = x_ref[...]
    o_ref[...] = (x - jnp.where(cols == lab, _M, 0.0)) * _S


def _tc_tail(sc_out, logits, labels_i32):
    b, c = logits.shape
    jblk = _C_SC // _TAIL_BLOCK
    block_b = 16
    labels2d = labels_i32.reshape(b, 1)
    return pl.pallas_call(
        _tail_body,
        grid=(b // block_b,),
        in_specs=[
            pl.BlockSpec(memory_space=pltpu.HBM),
            pl.BlockSpec((block_b, 1), lambda i: (i, 0)),
            pl.BlockSpec((block_b, _TAIL_BLOCK), lambda i: (i, jblk)),
        ],
        out_specs=pl.BlockSpec((block_b, _TAIL_BLOCK), lambda i: (i, jblk)),
        out_shape=jax.ShapeDtypeStruct((b, c), jnp.float32),
        input_output_aliases={0: 0},
    )(sc_out, labels2d, logits)


def kernel(logits, norms, labels):
    del norms
    b, _ = logits.shape
    labels_i32 = labels.astype(jnp.int32)
    labx = jnp.broadcast_to(labels_i32.reshape(b, 1), (b, 128))
    sc_out = _sc_scale(logits, labx)
    return _tc_tail(sc_out, logits, labels_i32)


# speedup vs baseline: 1.0324x; 1.0324x over previous
import jax
import jax.numpy as jnp
from jax import lax
from jax.experimental import pallas as pl
from jax.experimental.pallas import tpu as pltpu
from jax.experimental.pallas import tpu_sc as plsc

_S = 64.0
_M = 0.4

_NC = 2
_NS = 16
_NW = _NC * _NS
_CH = 1536
_NBUF = 4
_C_SC = 98304
_TAIL_BLOCK = 2048
_UNROLL = 16


def _sc_scale(logits, labx):
    b, c = logits.shape
    rows_per_w = b // _NW
    nch = _C_SC // _CH
    ngrp = rows_per_w // 8
    t_total = ngrp * nch

    def body(logits_ref, labx_ref, out_ref, labx_v, *scr):
        cid = lax.axis_index("c")
        sid = lax.axis_index("s")
        wid = sid * _NC + cid
        r0 = wid * rows_per_w
        pltpu.sync_copy(
            labx_ref.at[pl.ds(pl.multiple_of(r0, 8), rows_per_w)], labx_v)

        ins = scr[0:_NBUF]
        outs = scr[_NBUF:2 * _NBUF]
        lsems = scr[2 * _NBUF:3 * _NBUF]
        ssems = scr[3 * _NBUF:4 * _NBUF]

        def coords(t):
            rg = t // nch
            row8 = pl.multiple_of(r0 + rg * 8, 8)
            c0 = pl.multiple_of((t % nch) * _CH, 128)
            return rg, row8, c0

        def src_slice(t):
            _, row8, c0 = coords(t)
            return logits_ref.at[pl.ds(row8, 8), pl.ds(c0, _CH)]

        def dst_slice(t):
            _, row8, c0 = coords(t)
            return out_ref.at[pl.ds(row8, 8), pl.ds(c0, _CH)]

        for bb in range(_NBUF):
            pltpu.async_copy(src_slice(bb), ins[bb], lsems[bb])

        lanes = lax.broadcasted_iota(jnp.int32, (16,), 0)

        def group(g, carry):
            for bb in range(_NBUF):
                t = g * _NBUF + bb
                ib, ob, ls, ss = ins[bb], outs[bb], lsems[bb], ssems[bb]
                pltpu.make_async_copy(src_slice(t), ib, ls).wait()

                rg, _, c0 = coords(t)

                def row_fn(r, carry2, ib=ib, ob=ob, c0=c0, rg=rg):
                    labsplat = labx_v[rg * 8 + r, pl.ds(0, 16)]
                    lrel = labsplat - c0

                    def scale(i, carry3):
                        base = i * (16 * _UNROLL)
                        for u in range(_UNROLL):
                            sl = pl.ds(base + u * 16, 16)
                            m = (lrel - (base + u * 16)) == lanes
                            ob[r, sl] = (
                                ib[r, sl] - jnp.where(m, _M, 0.0)) * _S
                        return carry3

                    lax.fori_loop(0, _CH // (16 * _UNROLL), scale, 0)
                    return carry2

                lax.fori_loop(0, 8, row_fn, 0)

                @pl.when(t >= _NBUF)
                def _(ob=ob, ss=ss, t=t):
                    pltpu.make_async_copy(ob, dst_slice(t - _NBUF), ss).wait()

                pltpu.async_copy(ob, dst_slice(t), ss)

                @pl.when(t + _NBUF < t_total)
                def _(ib=ib, ls=ls, t=t):
                    pltpu.async_copy(src_slice(t + _NBUF), ib, ls)

            return carry

        lax.fori_loop(0, t_total // _NBUF, group, 0)
        for bb in range(_NBUF):
            pltpu.make_async_copy(
                outs[bb], dst_slice(t_total - _NBUF + bb), ssems[bb]).wait()

    mesh = plsc.VectorSubcoreMesh(
        core_axis_name="c", subcore_axis_name="s",
        num_cores=_NC, num_subcores=_NS,
    )
    fn = pl.kernel(
        body,
        out_type=jax.ShapeDtypeStruct((b, c), jnp.float32),
        mesh=mesh,
        scratch_types=(
            [pltpu.VMEM((rows_per_w, 128), jnp.int32)]
            + [pltpu.VMEM((8, _CH), jnp.float32)] * (2 * _NBUF)
            + [pltpu.SemaphoreType.DMA] * (2 * _NBUF)
        ),
    )
    return fn(logits, labx)


def _tail_body(alias_ref, labels_ref, x_ref, o_ref):
    del alias_ref
    bb, bc = x_ref.shape
    cols = _C_SC + jax.lax.broadcasted_iota(jnp.int32, (bb, bc), 1)
    lab = labels_ref[...]
    x = x_ref[...]
    o_ref[...] = (x - jnp.where(cols == lab, _M, 0.0)) * _S


def _tc_tail(sc_out, logits, labels_i32):
    b, c = logits.shape
    jblk = _C_SC // _TAIL_BLOCK
    block_b = 16
    labels2d = labels_i32.reshape(b, 1)
    return pl.pallas_call(
        _tail_body,
        grid=(b // block_b,),
        in_specs=[
            pl.BlockSpec(memory_space=pltpu.HBM),
            pl.BlockSpec((block_b, 1), lambda i: (i, 0)),
            pl.BlockSpec((block_b, _TAIL_BLOCK), lambda i: (i, jblk)),
        ],
        out_specs=pl.BlockSpec((block_b, _TAIL_BLOCK), lambda i: (i, jblk)),
        out_shape=jax.ShapeDtypeStruct((b, c), jnp.float32),
        input_output_aliases={0: 0},
    )(sc_out, labels2d, logits)


def kernel(logits, norms, labels):
    del norms
    b, _ = logits.shape
    labels_i32 = labels.astype(jnp.int32)
    labx = jnp.broadcast_to(labels_i32.reshape(b, 1), (b, 128))
    sc_out = _sc_scale(logits, labx)
    return _tc_tail(sc_out, logits, labels_i32)


# speedup vs baseline: 1.8959x; 1.8365x over previous
import jax
import jax.numpy as jnp
from jax import lax
from jax.experimental import pallas as pl
from jax.experimental.pallas import tpu as pltpu
from jax.experimental.pallas import tpu_sc as plsc

_S = 64.0
_M = 0.4

_NC = 2
_NS = 16
_NW = _NC * _NS
_CH = 1536
_NBUF = 4
_C_SC = 98304
_TAIL_BLOCK = 2048
_UNROLL = 16


def _sc_scale(logits, labx):
    b, c = logits.shape
    rows_per_w = b // _NW
    nch = _C_SC // _CH
    ngrp = rows_per_w // 8
    t_total = ngrp * nch

    def body(logits_ref, labx_ref, out_ref, labx_v, *scr):
        cid = lax.axis_index("c")
        sid = lax.axis_index("s")
        wid = sid * _NC + cid
        r0 = wid * rows_per_w
        pltpu.sync_copy(
            labx_ref.at[pl.ds(pl.multiple_of(r0, 8), rows_per_w)], labx_v)

        ins = scr[0:_NBUF]
        outs = scr[_NBUF:2 * _NBUF]
        lsems = scr[2 * _NBUF:3 * _NBUF]
        ssems = scr[3 * _NBUF:4 * _NBUF]

        def coords(t):
            rg = t // nch
            row8 = pl.multiple_of(r0 + rg * 8, 8)
            c0 = pl.multiple_of((t % nch) * _CH, 128)
            return rg, row8, c0

        def src_slice(t):
            _, row8, c0 = coords(t)
            return logits_ref.at[pl.ds(row8, 8), pl.ds(c0, _CH)]

        def dst_slice(t):
            _, row8, c0 = coords(t)
            return out_ref.at[pl.ds(row8, 8), pl.ds(c0, _CH)]

        for bb in range(_NBUF):
            pltpu.async_copy(src_slice(bb), ins[bb], lsems[bb])

        lanes = lax.broadcasted_iota(jnp.int32, (16,), 0)

        def group(g, carry):
            for bb in range(_NBUF):
                t = g * _NBUF + bb
                ib, ob, ls, ss = ins[bb], outs[bb], lsems[bb], ssems[bb]
                pltpu.make_async_copy(src_slice(t), ib, ls).wait()

                rg, _, c0 = coords(t)

                def chunk_compute(ib=ib, ob=ob, c0=c0, rg=rg):
                    @plsc.parallel_loop(0, 8)
                    def row_fn(r):
                        labsplat = labx_v[rg * 8 + r, pl.ds(0, 16)]
                        lrel = labsplat - c0

                        @plsc.parallel_loop(0, _CH // 16, unroll=_UNROLL)
                        def scale(i):
                            sl = pl.ds(i * 16, 16)
                            m = (lrel - i * 16) == lanes
                            ob[r, sl] = (
                                ib[r, sl] - jnp.where(m, _M, 0.0)) * _S

                chunk_compute()

                @pl.when(t >= _NBUF)
                def _(ob=ob, ss=ss, t=t):
                    pltpu.make_async_copy(ob, dst_slice(t - _NBUF), ss).wait()

                pltpu.async_copy(ob, dst_slice(t), ss)

                @pl.when(t + _NBUF < t_total)
                def _(ib=ib, ls=ls, t=t):
                    pltpu.async_copy(src_slice(t + _NBUF), ib, ls)

            return carry

        lax.fori_loop(0, t_total // _NBUF, group, 0)
        for bb in range(_NBUF):
            pltpu.make_async_copy(
                outs[bb], dst_slice(t_total - _NBUF + bb), ssems[bb]).wait()

    mesh = plsc.VectorSubcoreMesh(
        core_axis_name="c", subcore_axis_name="s",
        num_cores=_NC, num_subcores=_NS,
    )
    fn = pl.kernel(
        body,
        out_type=jax.ShapeDtypeStruct((b, c), jnp.float32),
        mesh=mesh,
        scratch_types=(
            [pltpu.VMEM((rows_per_w, 128), jnp.int32)]
            + [pltpu.VMEM((8, _CH), jnp.float32)] * (2 * _NBUF)
            + [pltpu.SemaphoreType.DMA] * (2 * _NBUF)
        ),
    )
    return fn(logits, labx)


def _tail_body(alias_ref, labels_ref, x_ref, o_ref):
    del alias_ref
    bb, bc = x_ref.shape
    cols = _C_SC + jax.lax.broadcasted_iota(jnp.int32, (bb, bc), 1)
    lab = labels_ref[...]
    x = x_ref[...]
    o_ref[...] = (x - jnp.where(cols == lab, _M, 0.0)) * _S


def _tc_tail(sc_out, logits, labels_i32):
    b, c = logits.shape
    jblk = _C_SC // _TAIL_BLOCK
    block_b = 16
    labels2d = labels_i32.reshape(b, 1)
    return pl.pallas_call(
        _tail_body,
        grid=(b // block_b,),
        in_specs=[
            pl.BlockSpec(memory_space=pltpu.HBM),
            pl.BlockSpec((block_b, 1), lambda i: (i, 0)),
            pl.BlockSpec((block_b, _TAIL_BLOCK), lambda i: (i, jblk)),
        ],
        out_specs=pl.BlockSpec((block_b, _TAIL_BLOCK), lambda i: (i, jblk)),
        out_shape=jax.ShapeDtypeStruct((b, c), jnp.float32),
        input_output_aliases={0: 0},
    )(sc_out, labels2d, logits)


def kernel(logits, norms, labels):
    del norms
    b, _ = logits.shape
    labels_i32 = labels.astype(jnp.int32)
    labx = jnp.broadcast_to(labels_i32.reshape(b, 1), (b, 128))
    sc_out = _sc_scale(logits, labx)
    return _tc_tail(sc_out, logits, labels_i32)
